# Initial kernel scaffold; baseline (speedup 1.0000x reference)
import jax
import jax.numpy as jnp
from jax.experimental import pallas as pl


def kernel(x, edge_index, batch, W1, b1, eps1, W2, b2, eps2, W3, b3, eps3, Wf, bf):
    raise NotImplementedError("write your pallas kernel here")


# speedup vs baseline: 9.5834x; 9.5834x over previous
import jax
import jax.numpy as jnp
from jax import lax
from jax.experimental import pallas as pl
from jax.experimental.pallas import tpu as pltpu
from jax.experimental.pallas import tpu_sc as plsc

N = 10000
D = 128
H = 64
G = 64
OUT = 10
E = 320000

NC = 2
NS = 16
NW = NC * NS
KC = 128
CH = 80
EPW = KC * CH
EPAD = NW * EPW
ACC_N = 10240
RPT = ACC_N // NS

ROWS_B = 2000
NB = N // ROWS_B


def _make_agg(width):
    def body(x_hbm, src_hbm, dst_hbm, zeros_hbm, out_hbm,
             src_v, dst_v, rows_v, acc_sh, sem):
        c = lax.axis_index("c")
        s = lax.axis_index("s")
        w = c * NS + s

        pltpu.sync_copy(zeros_hbm, acc_sh.at[pl.ds(s * RPT, RPT)])
        plsc.subcore_barrier()

        pltpu.sync_copy(src_hbm.at[w], src_v)
        pltpu.sync_copy(dst_hbm.at[w], dst_v)

        def chunk(j, carry):
            pltpu.async_copy(x_hbm.at[src_v.at[j]], rows_v, sem).wait()
            pltpu.sync_copy(rows_v, acc_sh.at[dst_v.at[j]], add=True)
            return carry

        lax.fori_loop(0, CH, chunk, 0)
        plsc.subcore_barrier()

        pltpu.sync_copy(acc_sh.at[pl.ds(s * RPT, RPT)],
                        out_hbm.at[c].at[pl.ds(s * RPT, RPT)])

    return pl.kernel(
        body,
        out_type=jax.ShapeDtypeStruct((NC, ACC_N, width), jnp.float32),
        mesh=plsc.VectorSubcoreMesh(core_axis_name="c", subcore_axis_name="s",
                                    num_cores=NC, num_subcores=NS),
        scratch_types=[
            pltpu.VMEM((CH, KC), jnp.int32),
            pltpu.VMEM((CH, KC), jnp.int32),
            pltpu.VMEM((KC, width), jnp.float32),
            pltpu.VMEM_SHARED((ACC_N, width), jnp.float32),
            pltpu.SemaphoreType.DMA,
        ],
        compiler_params=pltpu.CompilerParams(use_tc_tiling_on_sc=False),
    )


_agg_d = _make_agg(D)
_agg_h = _make_agg(H)


def _layer_body(scale_ref, x_ref, p_ref, b_ref, w_ref, o_ref):
    h = scale_ref[0, 0] * x_ref[...] + p_ref[0] + p_ref[1]
    o_ref[...] = jnp.tanh(
        jnp.dot(h, w_ref[...], preferred_element_type=jnp.float32)
        + b_ref[...])


def _layer(x, p, eps, W, b):
    win, wout = W.shape
    scale = (1.0 + eps).astype(jnp.float32).reshape(1, 1)
    return pl.pallas_call(
        _layer_body,
        grid=(NB,),
        in_specs=[
            pl.BlockSpec(memory_space=pltpu.SMEM),
            pl.BlockSpec((ROWS_B, win), lambda i: (i, 0)),
            pl.BlockSpec((NC, ROWS_B, win), lambda i: (0, i, 0)),
            pl.BlockSpec((1, wout), lambda i: (0, 0)),
            pl.BlockSpec((win, wout), lambda i: (0, 0)),
        ],
        out_specs=pl.BlockSpec((ROWS_B, wout), lambda i: (i, 0)),
        out_shape=jax.ShapeDtypeStruct((N, wout), jnp.float32),
    )(scale, x, p, b.reshape(1, wout), W)


def _final_body(scale_ref, x_ref, p_ref, b_ref, w_ref, bat_ref, wf_ref,
                bf_ref, o_ref, pooled_ref):
    i = pl.program_id(0)

    @pl.when(i == 0)
    def _zero():
        pooled_ref[...] = jnp.zeros_like(pooled_ref)

    hp = scale_ref[0, 0] * x_ref[...] + p_ref[0] + p_ref[1]
    h = jnp.tanh(
        jnp.dot(hp, w_ref[...], preferred_element_type=jnp.float32)
        + b_ref[...])
    bat = bat_ref[0, 0, :]
    onehot = (bat[None, :] ==
              lax.broadcasted_iota(jnp.int32, (G, ROWS_B), 0)
              ).astype(jnp.float32)
    pooled_ref[...] += jnp.dot(onehot, h, preferred_element_type=jnp.float32,
                               precision=lax.Precision.HIGHEST)

    @pl.when(i == pl.num_programs(0) - 1)
    def _emit():
        o_ref[...] = jnp.tanh(
            jnp.dot(pooled_ref[...], wf_ref[...],
                    preferred_element_type=jnp.float32) + bf_ref[...])


def _final(x, p, eps, W, b, batch3, Wf, bf):
    scale = (1.0 + eps).astype(jnp.float32).reshape(1, 1)
    return pl.pallas_call(
        _final_body,
        grid=(NB,),
        in_specs=[
            pl.BlockSpec(memory_space=pltpu.SMEM),
            pl.BlockSpec((ROWS_B, H), lambda i: (i, 0)),
            pl.BlockSpec((NC, ROWS_B, H), lambda i: (0, i, 0)),
            pl.BlockSpec((1, H), lambda i: (0, 0)),
            pl.BlockSpec((H, H), lambda i: (0, 0)),
            pl.BlockSpec((1, 1, ROWS_B), lambda i: (i, 0, 0)),
            pl.BlockSpec((H, OUT), lambda i: (0, 0)),
            pl.BlockSpec((1, OUT), lambda i: (0, 0)),
        ],
        out_specs=pl.BlockSpec((G, OUT), lambda i: (0, 0)),
        out_shape=jax.ShapeDtypeStruct((G, OUT), jnp.float32),
        scratch_shapes=[pltpu.VMEM((G, H), jnp.float32)],
    )(scale, x, p, b.reshape(1, H), W, batch3, Wf, bf.reshape(1, OUT))


def kernel(x, edge_index, batch, W1, b1, eps1, W2, b2, eps2, W3, b3, eps3,
           Wf, bf):
    src = edge_index[0]
    dst = edge_index[1]
    pad = EPAD - E
    pad_src = (jnp.arange(pad, dtype=jnp.int32) * 127) % N
    pad_dst = N + (jnp.arange(pad, dtype=jnp.int32) % (ACC_N - N))
    src3 = jnp.concatenate([src, pad_src]).reshape(NW, CH, KC)
    dst3 = jnp.concatenate([dst, pad_dst]).reshape(NW, CH, KC)
    zeros_d = jnp.zeros((RPT, D), jnp.float32)
    zeros_h = jnp.zeros((RPT, H), jnp.float32)
    batch3 = batch.reshape(NB, 1, ROWS_B)

    p = _agg_d(x, src3, dst3, zeros_d)
    h = _layer(x, p, eps1, W1, b1)
    p = _agg_h(h, src3, dst3, zeros_h)
    h = _layer(h, p, eps2, W2, b2)
    p = _agg_h(h, src3, dst3, zeros_h)
    return _final(h, p, eps3, W3, b3, batch3, Wf, bf)


# speedup vs baseline: 11.8373x; 1.2352x over previous
import jax
import jax.numpy as jnp
from jax import lax
from jax.experimental import pallas as pl
from jax.experimental.pallas import tpu as pltpu
from jax.experimental.pallas import tpu_sc as plsc

N = 10000
D = 128
H = 64
G = 64
OUT = 10
E = 320000

NC = 2
NS = 16
NW = NC * NS
KC = 128
CH = 80
EPW = KC * CH
EPAD = NW * EPW
ACC_N = 10240
RPT = ACC_N // NS

ROWS_B = 2000
NB = N // ROWS_B


def _make_agg(width, cstg):
    nstg = CH // cstg

    def body(x_hbm, src_hbm, dst_hbm, zeros_hbm, out_hbm,
             src_v, dst_v, rows_v, acc_sh, sem):
        c = lax.axis_index("c")
        s = lax.axis_index("s")
        w = c * NS + s
        gsem, ssem, isem = sem

        pltpu.sync_copy(zeros_hbm, acc_sh.at[pl.ds(s * RPT, RPT)])
        plsc.subcore_barrier()

        def stage_idx(st, b):
            return (pltpu.async_copy(
                        src_hbm.at[w].at[pl.ds(st * cstg, cstg)],
                        src_v.at[b], isem),
                    pltpu.async_copy(
                        dst_hbm.at[w].at[pl.ds(st * cstg, cstg)],
                        dst_v.at[b], isem))

        gathers = [None] * CH
        scatters = [None] * CH
        idx_cp = stage_idx(0, 0)
        for st in range(nstg):
            b = st % 2
            idx_cp[0].wait()
            idx_cp[1].wait()
            for k in range(cstg):
                j = st * cstg + k
                if j >= 2:
                    scatters[j - 2].wait()
                if k == 1 and st + 1 < nstg:
                    idx_cp = stage_idx(st + 1, 1 - b)
                gathers[j] = pltpu.async_copy(
                    x_hbm.at[src_v.at[b].at[k]], rows_v.at[j % 2], gsem)
                gathers[j].wait()
                scatters[j] = pltpu.async_copy(
                    rows_v.at[j % 2], acc_sh.at[dst_v.at[b].at[k]], ssem,
                    add=True)
        scatters[CH - 2].wait()
        scatters[CH - 1].wait()
        plsc.subcore_barrier()

        pltpu.sync_copy(acc_sh.at[pl.ds(s * RPT, RPT)],
                        out_hbm.at[c].at[pl.ds(s * RPT, RPT)])

    return pl.kernel(
        body,
        out_type=jax.ShapeDtypeStruct((NC, ACC_N, width), jnp.float32),
        mesh=plsc.VectorSubcoreMesh(core_axis_name="c", subcore_axis_name="s",
                                    num_cores=NC, num_subcores=NS),
        scratch_types=[
            pltpu.VMEM((2, cstg, KC), jnp.int32),
            pltpu.VMEM((2, cstg, KC), jnp.int32),
            pltpu.VMEM((2, KC, width), jnp.float32),
            pltpu.VMEM_SHARED((ACC_N, width), jnp.float32),
            (pltpu.SemaphoreType.DMA, pltpu.SemaphoreType.DMA,
             pltpu.SemaphoreType.DMA),
        ],
        compiler_params=pltpu.CompilerParams(use_tc_tiling_on_sc=False),
    )


_agg_d = _make_agg(D, 20)
_agg_h = _make_agg(H, 40)


def _layer_body(scale_ref, x_ref, p_ref, b_ref, w_ref, o_ref):
    h = scale_ref[0, 0] * x_ref[...] + p_ref[0] + p_ref[1]
    o_ref[...] = jnp.tanh(
        jnp.dot(h, w_ref[...], preferred_element_type=jnp.float32)
        + b_ref[...])


def _layer(x, p, eps, W, b):
    win, wout = W.shape
    scale = (1.0 + eps).astype(jnp.float32).reshape(1, 1)
    return pl.pallas_call(
        _layer_body,
        grid=(NB,),
        in_specs=[
            pl.BlockSpec(memory_space=pltpu.SMEM),
            pl.BlockSpec((ROWS_B, win), lambda i: (i, 0)),
            pl.BlockSpec((NC, ROWS_B, win), lambda i: (0, i, 0)),
            pl.BlockSpec((1, wout), lambda i: (0, 0)),
            pl.BlockSpec((win, wout), lambda i: (0, 0)),
        ],
        out_specs=pl.BlockSpec((ROWS_B, wout), lambda i: (i, 0)),
        out_shape=jax.ShapeDtypeStruct((N, wout), jnp.float32),
    )(scale, x, p, b.reshape(1, wout), W)


def _final_body(scale_ref, x_ref, p_ref, b_ref, w_ref, bat_ref, wf_ref,
                bf_ref, o_ref, pooled_ref):
    i = pl.program_id(0)

    @pl.when(i == 0)
    def _zero():
        pooled_ref[...] = jnp.zeros_like(pooled_ref)

    hp = scale_ref[0, 0] * x_ref[...] + p_ref[0] + p_ref[1]
    h = jnp.tanh(
        jnp.dot(hp, w_ref[...], preferred_element_type=jnp.float32)
        + b_ref[...])
    bat = bat_ref[0, 0, :]
    onehot = (bat[None, :] ==
              lax.broadcasted_iota(jnp.int32, (G, ROWS_B), 0)
              ).astype(jnp.float32)
    pooled_ref[...] += jnp.dot(onehot, h, preferred_element_type=jnp.float32,
                               precision=lax.Precision.HIGHEST)

    @pl.when(i == pl.num_programs(0) - 1)
    def _emit():
        o_ref[...] = jnp.tanh(
            jnp.dot(pooled_ref[...], wf_ref[...],
                    preferred_element_type=jnp.float32) + bf_ref[...])


def _final(x, p, eps, W, b, batch3, Wf, bf):
    scale = (1.0 + eps).astype(jnp.float32).reshape(1, 1)
    return pl.pallas_call(
        _final_body,
        grid=(NB,),
        in_specs=[
            pl.BlockSpec(memory_space=pltpu.SMEM),
            pl.BlockSpec((ROWS_B, H), lambda i: (i, 0)),
            pl.BlockSpec((NC, ROWS_B, H), lambda i: (0, i, 0)),
            pl.BlockSpec((1, H), lambda i: (0, 0)),
            pl.BlockSpec((H, H), lambda i: (0, 0)),
            pl.BlockSpec((1, 1, ROWS_B), lambda i: (i, 0, 0)),
            pl.BlockSpec((H, OUT), lambda i: (0, 0)),
            pl.BlockSpec((1, OUT), lambda i: (0, 0)),
        ],
        out_specs=pl.BlockSpec((G, OUT), lambda i: (0, 0)),
        out_shape=jax.ShapeDtypeStruct((G, OUT), jnp.float32),
        scratch_shapes=[pltpu.VMEM((G, H), jnp.float32)],
    )(scale, x, p, b.reshape(1, H), W, batch3, Wf, bf.reshape(1, OUT))


def kernel(x, edge_index, batch, W1, b1, eps1, W2, b2, eps2, W3, b3, eps3,
           Wf, bf):
    src = edge_index[0]
    dst = edge_index[1]
    pad = EPAD - E
    pad_src = (jnp.arange(pad, dtype=jnp.int32) * 127) % N
    pad_dst = N + (jnp.arange(pad, dtype=jnp.int32) % (ACC_N - N))
    src3 = jnp.concatenate([src, pad_src]).reshape(NW, CH, KC)
    dst3 = jnp.concatenate([dst, pad_dst]).reshape(NW, CH, KC)
    zeros_d = jnp.zeros((RPT, D), jnp.float32)
    zeros_h = jnp.zeros((RPT, H), jnp.float32)
    batch3 = batch.reshape(NB, 1, ROWS_B)

    p = _agg_d(x, src3, dst3, zeros_d)
    h = _layer(x, p, eps1, W1, b1)
    p = _agg_h(h, src3, dst3, zeros_h)
    h = _layer(h, p, eps2, W2, b2)
    p = _agg_h(h, src3, dst3, zeros_h)
    return _final(h, p, eps3, W3, b3, batch3, Wf, bf)


# speedup vs baseline: 14.6099x; 1.2342x over previous
import jax
import jax.numpy as jnp
from jax import lax
from jax.experimental import pallas as pl
from jax.experimental.pallas import tpu as pltpu
from jax.experimental.pallas import tpu_sc as plsc

N = 10000
D = 128
H = 64
G = 64
OUT = 10
E = 320000

NC = 2
NS = 16
NW = NC * NS
KC = 128
CH = 80
EPW = KC * CH
EPAD = NW * EPW
ACC_N = 10240
RPT = ACC_N // NS

ROWS_B = 2000
NB = N // ROWS_B


def _make_agg(width, cstg, dep):
    nstg = CH // cstg
    lead = dep - 2

    def body(x_hbm, src_hbm, dst_hbm, zeros_hbm, out_hbm,
             src_v, dst_v, rows_v, acc_sh, sem):
        c = lax.axis_index("c")
        s = lax.axis_index("s")
        w = c * NS + s
        gsem, ssem, isem = sem

        pltpu.sync_copy(zeros_hbm, acc_sh.at[pl.ds(s * RPT, RPT)])
        plsc.subcore_barrier()

        idx_desc = {}

        def issue_idx(st):
            b = st % 2
            idx_desc[st] = (
                pltpu.async_copy(src_hbm.at[w].at[pl.ds(st * cstg, cstg)],
                                 src_v.at[b], isem),
                pltpu.async_copy(dst_hbm.at[w].at[pl.ds(st * cstg, cstg)],
                                 dst_v.at[b], isem))

        idx_ready = set()

        def wait_idx(st):
            if st not in idx_ready:
                idx_desc[st][0].wait()
                idx_desc[st][1].wait()
                idx_ready.add(st)

        gathers = [None] * CH
        scatters = [None] * CH

        def start_gather(m):
            wait_idx(m // cstg)
            return pltpu.async_copy(
                x_hbm.at[src_v.at[(m // cstg) % 2].at[m % cstg]],
                rows_v.at[m % dep], gsem)

        issue_idx(0)
        for m in range(lead):
            gathers[m] = start_gather(m)
        for j in range(CH):
            m = j + lead
            if m < CH:
                if m - dep >= 0:
                    scatters[m - dep].wait()
                gathers[m] = start_gather(m)
            gathers[j].wait()
            scatters[j] = pltpu.async_copy(
                rows_v.at[j % dep],
                acc_sh.at[dst_v.at[(j // cstg) % 2].at[j % cstg]], ssem,
                add=True)
            if j % cstg == 1 and j // cstg + 1 < nstg:
                issue_idx(j // cstg + 1)
        for m in range(max(0, CH - dep), CH):
            scatters[m].wait()
        plsc.subcore_barrier()

        pltpu.sync_copy(acc_sh.at[pl.ds(s * RPT, RPT)],
                        out_hbm.at[c].at[pl.ds(s * RPT, RPT)])

    return pl.kernel(
        body,
        out_type=jax.ShapeDtypeStruct((NC, ACC_N, width), jnp.float32),
        mesh=plsc.VectorSubcoreMesh(core_axis_name="c", subcore_axis_name="s",
                                    num_cores=NC, num_subcores=NS),
        scratch_types=[
            pltpu.VMEM((2, cstg, KC), jnp.int32),
            pltpu.VMEM((2, cstg, KC), jnp.int32),
            pltpu.VMEM((dep, KC, width), jnp.float32),
            pltpu.VMEM_SHARED((ACC_N, width), jnp.float32),
            (pltpu.SemaphoreType.DMA, pltpu.SemaphoreType.DMA,
             pltpu.SemaphoreType.DMA),
        ],
        compiler_params=pltpu.CompilerParams(use_tc_tiling_on_sc=False),
    )


_agg_d = _make_agg(D, 20, 2)
_agg_h = _make_agg(H, 40, 4)


def _layer_body(scale_ref, x_ref, p_ref, b_ref, w_ref, o_ref):
    h = scale_ref[0, 0] * x_ref[...] + p_ref[0] + p_ref[1]
    o_ref[...] = jnp.tanh(
        jnp.dot(h, w_ref[...], preferred_element_type=jnp.float32)
        + b_ref[...])


def _layer(x, p, eps, W, b):
    win, wout = W.shape
    scale = (1.0 + eps).astype(jnp.float32).reshape(1, 1)
    return pl.pallas_call(
        _layer_body,
        grid=(NB,),
        in_specs=[
            pl.BlockSpec(memory_space=pltpu.SMEM),
            pl.BlockSpec((ROWS_B, win), lambda i: (i, 0)),
            pl.BlockSpec((NC, ROWS_B, win), lambda i: (0, i, 0)),
            pl.BlockSpec((1, wout), lambda i: (0, 0)),
            pl.BlockSpec((win, wout), lambda i: (0, 0)),
        ],
        out_specs=pl.BlockSpec((ROWS_B, wout), lambda i: (i, 0)),
        out_shape=jax.ShapeDtypeStruct((N, wout), jnp.float32),
    )(scale, x, p, b.reshape(1, wout), W)


def _final_body(scale_ref, x_ref, p_ref, b_ref, w_ref, bat_ref, wf_ref,
                bf_ref, o_ref, pooled_ref):
    i = pl.program_id(0)

    @pl.when(i == 0)
    def _zero():
        pooled_ref[...] = jnp.zeros_like(pooled_ref)

    hp = scale_ref[0, 0] * x_ref[...] + p_ref[0] + p_ref[1]
    h = jnp.tanh(
        jnp.dot(hp, w_ref[...], preferred_element_type=jnp.float32)
        + b_ref[...])
    bat = bat_ref[0, 0, :]
    onehot = (bat[None, :] ==
              lax.broadcasted_iota(jnp.int32, (G, ROWS_B), 0)
              ).astype(jnp.float32)
    pooled_ref[...] += jnp.dot(onehot, h, preferred_element_type=jnp.float32,
                               precision=lax.Precision.HIGHEST)

    @pl.when(i == pl.num_programs(0) - 1)
    def _emit():
        o_ref[...] = jnp.tanh(
            jnp.dot(pooled_ref[...], wf_ref[...],
                    preferred_element_type=jnp.float32) + bf_ref[...])


def _final(x, p, eps, W, b, batch3, Wf, bf):
    scale = (1.0 + eps).astype(jnp.float32).reshape(1, 1)
    return pl.pallas_call(
        _final_body,
        grid=(NB,),
        in_specs=[
            pl.BlockSpec(memory_space=pltpu.SMEM),
            pl.BlockSpec((ROWS_B, H), lambda i: (i, 0)),
            pl.BlockSpec((NC, ROWS_B, H), lambda i: (0, i, 0)),
            pl.BlockSpec((1, H), lambda i: (0, 0)),
            pl.BlockSpec((H, H), lambda i: (0, 0)),
            pl.BlockSpec((1, 1, ROWS_B), lambda i: (i, 0, 0)),
            pl.BlockSpec((H, OUT), lambda i: (0, 0)),
            pl.BlockSpec((1, OUT), lambda i: (0, 0)),
        ],
        out_specs=pl.BlockSpec((G, OUT), lambda i: (0, 0)),
        out_shape=jax.ShapeDtypeStruct((G, OUT), jnp.float32),
        scratch_shapes=[pltpu.VMEM((G, H), jnp.float32)],
    )(scale, x, p, b.reshape(1, H), W, batch3, Wf, bf.reshape(1, OUT))


def kernel(x, edge_index, batch, W1, b1, eps1, W2, b2, eps2, W3, b3, eps3,
           Wf, bf):
    src = edge_index[0]
    dst = edge_index[1]
    pad = EPAD - E
    pad_src = (jnp.arange(pad, dtype=jnp.int32) * 127) % N
    pad_dst = N + (jnp.arange(pad, dtype=jnp.int32) % (ACC_N - N))
    src3 = jnp.concatenate([src, pad_src]).reshape(NW, CH, KC)
    dst3 = jnp.concatenate([dst, pad_dst]).reshape(NW, CH, KC)
    zeros_d = jnp.zeros((RPT, D), jnp.float32)
    zeros_h = jnp.zeros((RPT, H), jnp.float32)
    batch3 = batch.reshape(NB, 1, ROWS_B)

    p = _agg_d(x, src3, dst3, zeros_d)
    h = _layer(x, p, eps1, W1, b1)
    p = _agg_h(h, src3, dst3, zeros_h)
    h = _layer(h, p, eps2, W2, b2)
    p = _agg_h(h, src3, dst3, zeros_h)
    return _final(h, p, eps3, W3, b3, batch3, Wf, bf)


# speedup vs baseline: 14.7043x; 1.0065x over previous
import jax
import jax.numpy as jnp
from jax import lax
from jax.experimental import pallas as pl
from jax.experimental.pallas import tpu as pltpu
from jax.experimental.pallas import tpu_sc as plsc

N = 10000
D = 128
H = 64
G = 64
OUT = 10
E = 320000

NC = 2
NS = 16
NW = NC * NS
KC = 128
CH = 80
EPW = KC * CH
EPAD = NW * EPW
ACC_N = 10240
RPT = ACC_N // NS

ROWS_B = 2000
NB = N // ROWS_B


def _make_agg(width, cstg, dep, lead, tc_tiling):
    nstg = CH // cstg
    assert 0 <= lead <= dep - 1 and lead < cstg
    k_pre = max(1, dep - lead)

    def body(x_hbm, src_hbm, dst_hbm, zeros_hbm, out_hbm,
             src_v, dst_v, rows_v, acc_sh, sem):
        c = lax.axis_index("c")
        s = lax.axis_index("s")
        w = c * NS + s
        gsem, ssem, isem = sem

        pltpu.sync_copy(zeros_hbm, acc_sh.at[pl.ds(s * RPT, RPT)])
        plsc.subcore_barrier()

        idx_desc = {}

        def issue_idx(st):
            b = st % 2
            idx_desc[st] = (
                pltpu.async_copy(src_hbm.at[w].at[pl.ds(st * cstg, cstg)],
                                 src_v.at[b], isem),
                pltpu.async_copy(dst_hbm.at[w].at[pl.ds(st * cstg, cstg)],
                                 dst_v.at[b], isem))

        idx_ready = set()

        def wait_idx(st):
            if st not in idx_ready:
                idx_desc[st][0].wait()
                idx_desc[st][1].wait()
                idx_ready.add(st)

        gathers = [None] * CH
        scatters = [None] * CH

        def start_gather(m):
            wait_idx(m // cstg)
            return pltpu.async_copy(
                x_hbm.at[src_v.at[(m // cstg) % 2].at[m % cstg]],
                rows_v.at[m % dep], gsem)

        issue_idx(0)
        for m in range(lead):
            gathers[m] = start_gather(m)
        for j in range(CH):
            m = j + lead
            if m < CH:
                if m - dep >= 0:
                    scatters[m - dep].wait()
                gathers[m] = start_gather(m)
            gathers[j].wait()
            scatters[j] = pltpu.async_copy(
                rows_v.at[j % dep],
                acc_sh.at[dst_v.at[(j // cstg) % 2].at[j % cstg]], ssem,
                add=True)
            if j % cstg == k_pre and j // cstg + 1 < nstg:
                issue_idx(j // cstg + 1)
        for m in range(max(0, CH - (dep - lead)), CH):
            scatters[m].wait()
        plsc.subcore_barrier()

        pltpu.sync_copy(acc_sh.at[pl.ds(s * RPT, RPT)],
                        out_hbm.at[c].at[pl.ds(s * RPT, RPT)])

    return pl.kernel(
        body,
        out_type=jax.ShapeDtypeStruct((NC, ACC_N, width), jnp.float32),
        mesh=plsc.VectorSubcoreMesh(core_axis_name="c", subcore_axis_name="s",
                                    num_cores=NC, num_subcores=NS),
        scratch_types=[
            pltpu.VMEM((2, cstg, KC), jnp.int32),
            pltpu.VMEM((2, cstg, KC), jnp.int32),
            pltpu.VMEM((dep, KC, width), jnp.float32),
            pltpu.VMEM_SHARED((ACC_N, width), jnp.float32),
            (pltpu.SemaphoreType.DMA, pltpu.SemaphoreType.DMA,
             pltpu.SemaphoreType.DMA),
        ],
        compiler_params=pltpu.CompilerParams(use_tc_tiling_on_sc=tc_tiling),
    )


_agg_d = _make_agg(D, 16, 2, 0, True)
_agg_h = _make_agg(H, 20, 8, 4, False)


def _layer_body(scale_ref, x_ref, p_ref, b_ref, w_ref, o_ref):
    h = scale_ref[0, 0] * x_ref[...] + p_ref[0] + p_ref[1]
    o_ref[...] = jnp.tanh(
        jnp.dot(h, w_ref[...], preferred_element_type=jnp.float32)
        + b_ref[...])


def _layer(x, p, eps, W, b):
    win, wout = W.shape
    scale = (1.0 + eps).astype(jnp.float32).reshape(1, 1)
    return pl.pallas_call(
        _layer_body,
        grid=(NB,),
        in_specs=[
            pl.BlockSpec(memory_space=pltpu.SMEM),
            pl.BlockSpec((ROWS_B, win), lambda i: (i, 0)),
            pl.BlockSpec((NC, ROWS_B, win), lambda i: (0, i, 0)),
            pl.BlockSpec((1, wout), lambda i: (0, 0)),
            pl.BlockSpec((win, wout), lambda i: (0, 0)),
        ],
        out_specs=pl.BlockSpec((ROWS_B, wout), lambda i: (i, 0)),
        out_shape=jax.ShapeDtypeStruct((N, wout), jnp.float32),
    )(scale, x, p, b.reshape(1, wout), W)


def _final_body(scale_ref, x_ref, p_ref, b_ref, w_ref, bat_ref, wf_ref,
                bf_ref, o_ref, pooled_ref):
    i = pl.program_id(0)

    @pl.when(i == 0)
    def _zero():
        pooled_ref[...] = jnp.zeros_like(pooled_ref)

    hp = scale_ref[0, 0] * x_ref[...] + p_ref[0] + p_ref[1]
    h = jnp.tanh(
        jnp.dot(hp, w_ref[...], preferred_element_type=jnp.float32)
        + b_ref[...])
    bat = bat_ref[0, 0, :]
    onehot = (bat[None, :] ==
              lax.broadcasted_iota(jnp.int32, (G, ROWS_B), 0)
              ).astype(jnp.float32)
    pooled_ref[...] += jnp.dot(onehot, h, preferred_element_type=jnp.float32,
                               precision=lax.Precision.HIGHEST)

    @pl.when(i == pl.num_programs(0) - 1)
    def _emit():
        o_ref[...] = jnp.tanh(
            jnp.dot(pooled_ref[...], wf_ref[...],
                    preferred_element_type=jnp.float32) + bf_ref[...])


def _final(x, p, eps, W, b, batch3, Wf, bf):
    scale = (1.0 + eps).astype(jnp.float32).reshape(1, 1)
    return pl.pallas_call(
        _final_body,
        grid=(NB,),
        in_specs=[
            pl.BlockSpec(memory_space=pltpu.SMEM),
            pl.BlockSpec((ROWS_B, H), lambda i: (i, 0)),
            pl.BlockSpec((NC, ROWS_B, H), lambda i: (0, i, 0)),
            pl.BlockSpec((1, H), lambda i: (0, 0)),
            pl.BlockSpec((H, H), lambda i: (0, 0)),
            pl.BlockSpec((1, 1, ROWS_B), lambda i: (i, 0, 0)),
            pl.BlockSpec((H, OUT), lambda i: (0, 0)),
            pl.BlockSpec((1, OUT), lambda i: (0, 0)),
        ],
        out_specs=pl.BlockSpec((G, OUT), lambda i: (0, 0)),
        out_shape=jax.ShapeDtypeStruct((G, OUT), jnp.float32),
        scratch_shapes=[pltpu.VMEM((G, H), jnp.float32)],
    )(scale, x, p, b.reshape(1, H), W, batch3, Wf, bf.reshape(1, OUT))


def kernel(x, edge_index, batch, W1, b1, eps1, W2, b2, eps2, W3, b3, eps3,
           Wf, bf):
    src = edge_index[0]
    dst = edge_index[1]
    pad = EPAD - E
    pad_src = (jnp.arange(pad, dtype=jnp.int32) * 127) % N
    pad_dst = N + (jnp.arange(pad, dtype=jnp.int32) % (ACC_N - N))
    src3 = jnp.concatenate([src, pad_src]).reshape(NW, CH, KC)
    dst3 = jnp.concatenate([dst, pad_dst]).reshape(NW, CH, KC)
    zeros_d = jnp.zeros((RPT, D), jnp.float32)
    zeros_h = jnp.zeros((RPT, H), jnp.float32)
    batch3 = batch.reshape(NB, 1, ROWS_B)

    p = _agg_d(x, src3, dst3, zeros_d)
    h = _layer(x, p, eps1, W1, b1)
    p = _agg_h(h, src3, dst3, zeros_h)
    h = _layer(h, p, eps2, W2, b2)
    p = _agg_h(h, src3, dst3, zeros_h)
    return _final(h, p, eps3, W3, b3, batch3, Wf, bf)


# speedup vs baseline: 15.8074x; 1.0750x over previous
import jax
import jax.numpy as jnp
from jax import lax
from jax.experimental import pallas as pl
from jax.experimental.pallas import tpu as pltpu
from jax.experimental.pallas import tpu_sc as plsc

N = 10000
D = 128
H = 64
G = 64
OUT = 10
E = 320000

NC = 2
NS = 16
NW = NC * NS
KC = 128
CH = 80
EPW = KC * CH
EPAD = NW * EPW
ACC_N = 10240
RPT = ACC_N // NS

ROWS_B = 2000
NB = N // ROWS_B


def _make_agg(width, cstg, dep, lead, tc_tiling, nch, split):
    nstg = nch // cstg
    assert 0 <= lead <= dep - 1 and lead < cstg
    k_pre = max(1, dep - lead)

    def body(x_hbm, src_hbm, dst_hbm, zeros_hbm, out_hbm,
             src_v, dst_v, rows_v, acc_sh, sem):
        c = lax.axis_index("c")
        s = lax.axis_index("s")
        w = c * NS + s
        gsem, ssem, isem = sem
        if split:
            src_view = src_hbm.at[c].at[s]
            dst_view = dst_hbm.at[s]
        else:
            src_view = src_hbm.at[w]
            dst_view = dst_hbm.at[w]

        pltpu.sync_copy(zeros_hbm, acc_sh.at[pl.ds(s * RPT, RPT)])
        plsc.subcore_barrier()

        idx_desc = {}

        def issue_idx(st):
            b = st % 2
            idx_desc[st] = (
                pltpu.async_copy(src_view.at[pl.ds(st * cstg, cstg)],
                                 src_v.at[b], isem),
                pltpu.async_copy(dst_view.at[pl.ds(st * cstg, cstg)],
                                 dst_v.at[b], isem))

        idx_ready = set()

        def wait_idx(st):
            if st not in idx_ready:
                idx_desc[st][0].wait()
                idx_desc[st][1].wait()
                idx_ready.add(st)

        gathers = [None] * nch
        scatters = [None] * nch

        def start_gather(m):
            wait_idx(m // cstg)
            return pltpu.async_copy(
                x_hbm.at[src_v.at[(m // cstg) % 2].at[m % cstg]],
                rows_v.at[m % dep], gsem)

        issue_idx(0)
        for m in range(lead):
            gathers[m] = start_gather(m)
        for j in range(nch):
            m = j + lead
            if m < nch:
                if m - dep >= 0:
                    scatters[m - dep].wait()
                gathers[m] = start_gather(m)
            gathers[j].wait()
            scatters[j] = pltpu.async_copy(
                rows_v.at[j % dep],
                acc_sh.at[dst_v.at[(j // cstg) % 2].at[j % cstg]], ssem,
                add=True)
            if j % cstg == k_pre and j // cstg + 1 < nstg:
                issue_idx(j // cstg + 1)
        for m in range(max(0, nch - (dep - lead)), nch):
            scatters[m].wait()
        plsc.subcore_barrier()

        pltpu.sync_copy(acc_sh.at[pl.ds(s * RPT, RPT)],
                        out_hbm.at[c].at[pl.ds(s * RPT, RPT)])

    return pl.kernel(
        body,
        out_type=jax.ShapeDtypeStruct((NC, ACC_N, width), jnp.float32),
        mesh=plsc.VectorSubcoreMesh(core_axis_name="c", subcore_axis_name="s",
                                    num_cores=NC, num_subcores=NS),
        scratch_types=[
            pltpu.VMEM((2, cstg, KC), jnp.int32),
            pltpu.VMEM((2, cstg, KC), jnp.int32),
            pltpu.VMEM((dep, KC, width), jnp.float32),
            pltpu.VMEM_SHARED((ACC_N, width), jnp.float32),
            (pltpu.SemaphoreType.DMA, pltpu.SemaphoreType.DMA,
             pltpu.SemaphoreType.DMA),
        ],
        compiler_params=pltpu.CompilerParams(use_tc_tiling_on_sc=tc_tiling),
    )


CH2 = EPAD // (NS * KC)
_agg_d = _make_agg(H, 20, 8, 4, False, CH2, True)
_agg_h = _make_agg(H, 20, 8, 4, False, CH, False)


def _layer1_body(scale_ref, x_ref, p_ref, b_ref, w_ref, o_ref):
    agg = jnp.concatenate([p_ref[0], p_ref[1]], axis=1)
    h = scale_ref[0, 0] * x_ref[...] + agg
    o_ref[...] = jnp.tanh(
        jnp.dot(h, w_ref[...], preferred_element_type=jnp.float32)
        + b_ref[...])


def _layer1(x, p, eps, W, b):
    scale = (1.0 + eps).astype(jnp.float32).reshape(1, 1)
    return pl.pallas_call(
        _layer1_body,
        grid=(NB,),
        in_specs=[
            pl.BlockSpec(memory_space=pltpu.SMEM),
            pl.BlockSpec((ROWS_B, D), lambda i: (i, 0)),
            pl.BlockSpec((NC, ROWS_B, H), lambda i: (0, i, 0)),
            pl.BlockSpec((1, H), lambda i: (0, 0)),
            pl.BlockSpec((D, H), lambda i: (0, 0)),
        ],
        out_specs=pl.BlockSpec((ROWS_B, H), lambda i: (i, 0)),
        out_shape=jax.ShapeDtypeStruct((N, H), jnp.float32),
    )(scale, x, p, b.reshape(1, H), W)


def _layer_body(scale_ref, x_ref, p_ref, b_ref, w_ref, o_ref):
    h = scale_ref[0, 0] * x_ref[...] + p_ref[0] + p_ref[1]
    o_ref[...] = jnp.tanh(
        jnp.dot(h, w_ref[...], preferred_element_type=jnp.float32)
        + b_ref[...])


def _layer(x, p, eps, W, b):
    win, wout = W.shape
    scale = (1.0 + eps).astype(jnp.float32).reshape(1, 1)
    return pl.pallas_call(
        _layer_body,
        grid=(NB,),
        in_specs=[
            pl.BlockSpec(memory_space=pltpu.SMEM),
            pl.BlockSpec((ROWS_B, win), lambda i: (i, 0)),
            pl.BlockSpec((NC, ROWS_B, win), lambda i: (0, i, 0)),
            pl.BlockSpec((1, wout), lambda i: (0, 0)),
            pl.BlockSpec((win, wout), lambda i: (0, 0)),
        ],
        out_specs=pl.BlockSpec((ROWS_B, wout), lambda i: (i, 0)),
        out_shape=jax.ShapeDtypeStruct((N, wout), jnp.float32),
    )(scale, x, p, b.reshape(1, wout), W)


def _final_body(scale_ref, x_ref, p_ref, b_ref, w_ref, bat_ref, wf_ref,
                bf_ref, o_ref, pooled_ref):
    i = pl.program_id(0)

    @pl.when(i == 0)
    def _zero():
        pooled_ref[...] = jnp.zeros_like(pooled_ref)

    hp = scale_ref[0, 0] * x_ref[...] + p_ref[0] + p_ref[1]
    h = jnp.tanh(
        jnp.dot(hp, w_ref[...], preferred_element_type=jnp.float32)
        + b_ref[...])
    bat = bat_ref[0, 0, :]
    onehot = (bat[None, :] ==
              lax.broadcasted_iota(jnp.int32, (G, ROWS_B), 0)
              ).astype(jnp.float32)
    pooled_ref[...] += jnp.dot(onehot, h, preferred_element_type=jnp.float32,
                               precision=lax.Precision.HIGHEST)

    @pl.when(i == pl.num_programs(0) - 1)
    def _emit():
        o_ref[...] = jnp.tanh(
            jnp.dot(pooled_ref[...], wf_ref[...],
                    preferred_element_type=jnp.float32) + bf_ref[...])


def _final(x, p, eps, W, b, batch3, Wf, bf):
    scale = (1.0 + eps).astype(jnp.float32).reshape(1, 1)
    return pl.pallas_call(
        _final_body,
        grid=(NB,),
        in_specs=[
            pl.BlockSpec(memory_space=pltpu.SMEM),
            pl.BlockSpec((ROWS_B, H), lambda i: (i, 0)),
            pl.BlockSpec((NC, ROWS_B, H), lambda i: (0, i, 0)),
            pl.BlockSpec((1, H), lambda i: (0, 0)),
            pl.BlockSpec((H, H), lambda i: (0, 0)),
            pl.BlockSpec((1, 1, ROWS_B), lambda i: (i, 0, 0)),
            pl.BlockSpec((H, OUT), lambda i: (0, 0)),
            pl.BlockSpec((1, OUT), lambda i: (0, 0)),
        ],
        out_specs=pl.BlockSpec((G, OUT), lambda i: (0, 0)),
        out_shape=jax.ShapeDtypeStruct((G, OUT), jnp.float32),
        scratch_shapes=[pltpu.VMEM((G, H), jnp.float32)],
    )(scale, x, p, b.reshape(1, H), W, batch3, Wf, bf.reshape(1, OUT))


def kernel(x, edge_index, batch, W1, b1, eps1, W2, b2, eps2, W3, b3, eps3,
           Wf, bf):
    src = edge_index[0]
    dst = edge_index[1]
    pad = EPAD - E
    pad_src = (jnp.arange(pad, dtype=jnp.int32) * 127) % N
    pad_dst = N + (jnp.arange(pad, dtype=jnp.int32) % (ACC_N - N))
    sb = jnp.concatenate([src, pad_src])
    db = jnp.concatenate([dst, pad_dst])
    src3 = sb.reshape(NW, CH, KC)
    dst3 = db.reshape(NW, CH, KC)
    src4 = jnp.stack([2 * sb, 2 * sb + 1]).reshape(NC, NS, CH2, KC)
    dst4 = db.reshape(NS, CH2, KC)
    zeros_h = jnp.zeros((RPT, H), jnp.float32)
    batch3 = batch.reshape(NB, 1, ROWS_B)

    p = _agg_d(x.reshape(2 * N, H), src4, dst4, zeros_h)
    h = _layer1(x, p, eps1, W1, b1)
    p = _agg_h(h, src3, dst3, zeros_h)
    h = _layer(h, p, eps2, W2, b2)
    p = _agg_h(h, src3, dst3, zeros_h)
    return _final(h, p, eps3, W3, b3, batch3, Wf, bf)
